# baseline (speedup 1.0000x reference)
import functools

import jax
import jax.numpy as jnp
from jax import lax
from jax.experimental import pallas as pl
from jax.experimental.pallas import tpu as pltpu
from jax.experimental.pallas import tpu_sc as plsc

N_E = 8192
E_DIM = 256
BETA = 0.25

TT = 2048
NC = 512


WINDOWS = ((0, 2736), (2736, 5472), (5472, N_E))


def _bf16_rt(x):
    return x.astype(jnp.bfloat16).astype(jnp.float32)


def _dist_body(z_ref, z2_ref, e_ref, e2_ref, idx_ref, loss_ref, acc_ref):
    step = pl.program_id(0)
    nsteps = pl.num_programs(0)
    z = z_ref[...]
    zb = z.astype(jnp.bfloat16)
    z2 = z2_ref[...]

    wins = []
    for lo, hi in WINDOWS:
        rmin = jnp.full((TT, 1), jnp.inf, jnp.float32)
        ridx = jnp.zeros((TT, 1), jnp.int32)
        start = lo
        while start < hi:
            size = min(NC, hi - start)
            eb = e_ref[pl.ds(start, size), :].astype(jnp.bfloat16)
            m = lax.dot_general(zb, eb, (((1,), (1,)), ((), ())),
                                preferred_element_type=jnp.float32)
            d = (z2 + e2_ref[:, pl.ds(start, size)]) - 2.0 * m
            cmin = jnp.min(d, axis=1, keepdims=True)
            gidx = lax.broadcasted_iota(jnp.int32, (TT, size), 1) + start
            cidx = jnp.min(jnp.where(d == cmin, gidx, N_E), axis=1,
                           keepdims=True)
            upd = cmin < rmin
            rmin = jnp.where(upd, cmin, rmin)
            ridx = jnp.where(upd, cidx, ridx)
            start += size
        wins.append((rmin, ridx))

    (m0, i0), (m1, i1), (m2, i2) = wins
    curm = _bf16_rt(m0)
    curi = i0
    curv = m0
    for mw, iw in ((m1, i1), (m2, i2)):
        take = (mw < curm) | ((mw == curm) & (iw < curi))
        curv = jnp.where(take, mw, curv)
        curi = jnp.where(take, iw, curi)
        curm = _bf16_rt(jnp.where(take, mw, curm))
    idx_ref[...] = curi
    run_min = curv

    @pl.when(step == 0)
    def _():
        acc_ref[0] = 0.0
    acc_ref[0] += jnp.sum(run_min)

    @pl.when(step == nsteps - 1)
    def _():
        mean_sq = acc_ref[0] / (nsteps * TT * E_DIM)
        loss_ref[...] = jnp.full((1, 1), mean_sq + BETA * mean_sq, jnp.float32)


def _distance_argmin(z_flat, z2, emb, e2):
    tok = z_flat.shape[0]
    grid = (tok // TT,)
    return pl.pallas_call(
        _dist_body,
        grid=grid,
        in_specs=[
            pl.BlockSpec((TT, E_DIM), lambda i: (i, 0)),
            pl.BlockSpec((TT, 1), lambda i: (i, 0)),
            pl.BlockSpec((N_E, E_DIM), lambda i: (0, 0)),
            pl.BlockSpec((1, N_E), lambda i: (0, 0)),
        ],
        out_specs=[
            pl.BlockSpec((TT, 1), lambda i: (i, 0)),
            pl.BlockSpec((1, 1), lambda i: (0, 0)),
        ],
        out_shape=[
            jax.ShapeDtypeStruct((tok, 1), jnp.int32),
            jax.ShapeDtypeStruct((1, 1), jnp.float32),
        ],
        scratch_shapes=[pltpu.SMEM((1,), jnp.float32)],
        compiler_params=pltpu.CompilerParams(
            dimension_semantics=("arbitrary",)),
    )(z_flat, z2, emb, e2)


def _sc_gather(emb, idx):
    tok = idx.shape[0]
    info = plsc.get_sparse_core_info()
    ncores, nsub = info.num_cores, info.num_subcores
    nw = ncores * nsub
    b_per_w = tok // nw
    ch = 128
    nch = b_per_w // ch
    idx2d = idx.reshape(tok // ch, ch)

    mesh = plsc.VectorSubcoreMesh(core_axis_name="c", subcore_axis_name="s")

    @functools.partial(
        pl.kernel, mesh=mesh,
        out_type=jax.ShapeDtypeStruct((tok, E_DIM), jnp.float32),
        scratch_types=[
            pltpu.VMEM((nch, ch), jnp.int32),
            pltpu.VMEM((ch, E_DIM), jnp.float32),
            pltpu.SemaphoreType.DMA,
        ],
    )
    def k(table_hbm, idx_hbm, out_hbm, idx_v, rows_v, sem):
        wid = lax.axis_index("s") * ncores + lax.axis_index("c")
        base = wid * b_per_w
        pltpu.sync_copy(idx_hbm.at[pl.ds(wid * nch, nch)], idx_v)
        for c in range(nch):
            pltpu.async_copy(table_hbm.at[idx_v.at[c]], rows_v, sem).wait()
            pltpu.sync_copy(rows_v, out_hbm.at[pl.ds(base + c * ch, ch)])

    return k(emb, idx2d)


def kernel(z, embedding_weight):
    B, D, T = z.shape
    zp = jnp.transpose(z, (0, 2, 1))
    z_flat = zp.reshape(-1, D)
    z2 = jnp.sum(z_flat ** 2, axis=1, keepdims=True)
    e2 = jnp.sum(embedding_weight ** 2, axis=1).reshape(1, N_E)
    idx2d, loss = _distance_argmin(z_flat, z2, embedding_weight, e2)
    idx = idx2d.reshape(-1)
    rows = _sc_gather(embedding_weight, idx)
    z_q_out = jnp.transpose(rows.reshape(B, T, D), (0, 2, 1))
    return z_q_out, loss[0, 0], idx2d.reshape(B, T)
